# Initial kernel scaffold; baseline (speedup 1.0000x reference)
import jax
import jax.numpy as jnp
from jax.experimental import pallas as pl


def kernel(nodes, edge_index, batch_index, emb, W1, b1, W2, b2):
    raise NotImplementedError("write your pallas kernel here")


# speedup vs baseline: 25.0611x; 25.0611x over previous
import functools

import jax
import jax.numpy as jnp
from jax import lax
from jax.experimental import pallas as pl
from jax.experimental.pallas import tpu as pltpu
from jax.experimental.pallas import tpu_sc as plsc

N = 10000
E = 320000
D = 128
VP = 32
G = 64
NP = 10240
CZ = NP * VP
NC = 2
NS = 16
NT = NC * NS
PERT = E // NT
CH = 128
NFULL = PERT // CH
TAIL = PERT - NFULL * CH

f32 = jnp.float32
i32 = jnp.int32


def _mesh():
    return plsc.VectorSubcoreMesh(
        core_axis_name="c", subcore_axis_name="s", num_cores=NC, num_subcores=NS
    )


_SC_PARAMS = pltpu.CompilerParams(needs_layout_passes=False)


@functools.partial(
    pl.kernel,
    out_type=jax.ShapeDtypeStruct((NC, NP), f32),
    mesh=_mesh(),
    compiler_params=_SC_PARAMS,
    scratch_types=[
        pltpu.VMEM((CH,), i32),
        pltpu.VMEM((CH,), f32),
        pltpu.VMEM((16,), i32),
        pltpu.VMEM((16,), f32),
        pltpu.VMEM_SHARED((NP,), f32),
    ],
)
def _sc_degree(dst_hbm, z1_hbm, out_hbm, idx_b, ones_b, idxt_b, onest_b, deg_sp):
    c = lax.axis_index("c")
    s = lax.axis_index("s")
    for j in range(CH // 16):
        ones_b[pl.ds(j * 16, 16)] = jnp.full((16,), 1.0, f32)
    onest_b[...] = jnp.full((16,), 1.0, f32)
    sl = NP // NS
    pltpu.sync_copy(z1_hbm.at[pl.ds(0, sl)], deg_sp.at[pl.ds(s * sl, sl)])
    plsc.subcore_barrier()
    base = (c * NS + s) * PERT

    def body(i, _):
        pltpu.sync_copy(dst_hbm.at[pl.ds(base + i * CH, CH)], idx_b)
        pltpu.sync_copy(ones_b, deg_sp.at[idx_b], add=True)
        return 0

    lax.fori_loop(0, NFULL, body, 0)
    pltpu.sync_copy(dst_hbm.at[pl.ds(base + NFULL * CH, TAIL)], idxt_b)
    pltpu.sync_copy(onest_b, deg_sp.at[idxt_b], add=True)
    plsc.subcore_barrier()
    pltpu.sync_copy(deg_sp.at[pl.ds(s * sl, sl)], out_hbm.at[c, pl.ds(s * sl, sl)])


@functools.partial(
    pl.kernel,
    out_type=jax.ShapeDtypeStruct((NC, CZ), f32),
    mesh=_mesh(),
    compiler_params=_SC_PARAMS,
    scratch_types=[
        pltpu.VMEM((NP,), i32),
        pltpu.VMEM((NP,), f32),
        pltpu.VMEM((CH,), i32),
        pltpu.VMEM((CH,), i32),
        pltpu.VMEM((CH,), i32),
        pltpu.VMEM((CH,), f32),
        pltpu.VMEM((16,), i32),
        pltpu.VMEM((16,), i32),
        pltpu.VMEM((16,), i32),
        pltpu.VMEM((16,), f32),
        pltpu.VMEM_SHARED((CZ,), f32),
    ],
)
def _sc_hist(src_hbm, dst_hbm, nodes_hbm, dinv_hbm, z1_hbm, out_hbm,
             nodes_v, dinv_v, srcb, dstb, idxb, valb,
             srct, dstt, idxt, valt, c_sp):
    c = lax.axis_index("c")
    s = lax.axis_index("s")
    pltpu.sync_copy(nodes_hbm, nodes_v)
    pltpu.sync_copy(dinv_hbm, dinv_v)
    sl = CZ // NS
    pltpu.sync_copy(z1_hbm.at[pl.ds(0, sl)], c_sp.at[pl.ds(s * sl, sl)])
    plsc.subcore_barrier()
    base = (c * NS + s) * PERT

    def body(i, _):
        off = base + i * CH
        pltpu.sync_copy(src_hbm.at[pl.ds(off, CH)], srcb)
        pltpu.sync_copy(dst_hbm.at[pl.ds(off, CH)], dstb)
        for j in range(CH // 16):
            sv = srcb[pl.ds(j * 16, 16)]
            dv = dstb[pl.ds(j * 16, 16)]
            v = plsc.load_gather(nodes_v, [sv])
            w = plsc.load_gather(dinv_v, [sv])
            idxb[pl.ds(j * 16, 16)] = dv * VP + v
            valb[pl.ds(j * 16, 16)] = w
        pltpu.sync_copy(valb, c_sp.at[idxb], add=True)
        return 0

    lax.fori_loop(0, NFULL, body, 0)
    off = base + NFULL * CH
    pltpu.sync_copy(src_hbm.at[pl.ds(off, TAIL)], srct)
    pltpu.sync_copy(dst_hbm.at[pl.ds(off, TAIL)], dstt)
    sv = srct[...]
    dv = dstt[...]
    v = plsc.load_gather(nodes_v, [sv])
    w = plsc.load_gather(dinv_v, [sv])
    idxt[...] = dv * VP + v
    valt[...] = w
    pltpu.sync_copy(valt, c_sp.at[idxt], add=True)
    plsc.subcore_barrier()
    pltpu.sync_copy(c_sp.at[pl.ds(s * sl, sl)], out_hbm.at[c, pl.ds(s * sl, sl)])


@functools.partial(
    pl.kernel,
    out_type=jax.ShapeDtypeStruct((NC, NP, D), f32),
    mesh=_mesh(),
    compiler_params=_SC_PARAMS,
    scratch_types=[
        pltpu.VMEM((CH,), i32),
        pltpu.VMEM((CH,), i32),
        pltpu.VMEM((CH,), i32),
        pltpu.VMEM((CH,), i32),
        pltpu.VMEM((CH, D), f32),
        pltpu.VMEM((CH, D), f32),
        pltpu.SemaphoreType.DMA,
        pltpu.SemaphoreType.DMA,
        pltpu.SemaphoreType.DMA,
        pltpu.SemaphoreType.DMA,
        pltpu.VMEM((16,), i32),
        pltpu.VMEM((16,), i32),
        pltpu.VMEM((16, D), f32),
        pltpu.VMEM_SHARED((NP, D), f32),
    ],
)
def _sc_segsum(src_hbm, dst_hbm, y_hbm, z2_hbm, out_hbm,
               sidx0, sidx1, didx0, didx1, rows0, rows1,
               gsem0, gsem1, ssem0, ssem1,
               sidxt, didxt, rowst, s_sp):
    c = lax.axis_index("c")
    s = lax.axis_index("s")
    sl = NP // NS
    pltpu.sync_copy(z2_hbm.at[pl.ds(s * sl, sl)], s_sp.at[pl.ds(s * sl, sl)])
    plsc.subcore_barrier()
    base = (c * NS + s) * PERT

    sidx = (sidx0, sidx1)
    didx = (didx0, didx1)
    rows = (rows0, rows1)
    gsem = (gsem0, gsem1)
    ssem = (ssem0, ssem1)

    def load_idx(g, b):
        off = base + g * CH
        pltpu.sync_copy(src_hbm.at[pl.ds(off, CH)], sidx[b])
        pltpu.sync_copy(dst_hbm.at[pl.ds(off, CH)], didx[b])

    def start_gather(b):
        pltpu.async_copy(y_hbm.at[sidx[b]], rows[b], gsem[b])

    def wait_gather(b):
        pltpu.make_async_copy(y_hbm.at[sidx[b]], rows[b], gsem[b]).wait()

    def start_scatter(b):
        pltpu.async_copy(rows[b], s_sp.at[didx[b]], ssem[b], add=True)

    def wait_scatter(b):
        pltpu.make_async_copy(rows[b], s_sp.at[didx[b]], ssem[b]).wait()

    load_idx(0, 0)
    start_gather(0)
    load_idx(1, 1)
    start_gather(1)

    def body(k, _):
        for b in range(2):
            g = 2 * k + b
            wait_gather(b)
            start_scatter(b)
            n = g + 2

            @pl.when(n < NFULL)
            def _issue_next():
                wait_scatter(b)
                load_idx(n, b)
                start_gather(b)

        return 0

    lax.fori_loop(0, NFULL // 2, body, 0)
    wait_scatter(0)
    wait_scatter(1)
    off = base + NFULL * CH
    pltpu.sync_copy(src_hbm.at[pl.ds(off, TAIL)], sidxt)
    pltpu.sync_copy(dst_hbm.at[pl.ds(off, TAIL)], didxt)
    pltpu.sync_copy(y_hbm.at[sidxt], rowst)
    pltpu.sync_copy(rowst, s_sp.at[didxt], add=True)
    plsc.subcore_barrier()
    pltpu.sync_copy(s_sp.at[pl.ds(s * sl, sl)], out_hbm.at[c, pl.ds(s * sl, sl)])


def _tc_prep_body(degp_ref, emb_ref, w1_ref, dinv_ref, t1p_ref):
    deg = degp_ref[0] + degp_ref[1] + 1.0
    dinv_ref[...] = lax.rsqrt(deg)
    t1p_ref[...] = jnp.dot(emb_ref[...], w1_ref[...], preferred_element_type=f32)


def _tc_prep(deg_p, emb_pad, w1):
    return pl.pallas_call(
        _tc_prep_body,
        out_shape=(
            jax.ShapeDtypeStruct((NP // 128, 128), f32),
            jax.ShapeDtypeStruct((VP, D), f32),
        ),
    )(deg_p, emb_pad, w1)


_B = 512
_NB = NP // _B


def _tc_l1_body(cp_ref, nodes_ref, dinv_ref, t1p_ref, b1_ref, y_ref):
    cmat = cp_ref[0] + cp_ref[1]
    dinv = dinv_ref[...]
    oh = jnp.where(
        nodes_ref[...] == lax.broadcasted_iota(i32, (_B, VP), 1), dinv, 0.0
    )
    cmat = cmat + oh
    xw = jnp.dot(cmat, t1p_ref[...], preferred_element_type=f32)
    x2 = jnp.maximum(dinv * xw + b1_ref[...], 0.0)
    y_ref[...] = dinv * x2


def _tc_layer1(c_p, nodes_col, dinv_col, t1p, b1r):
    return pl.pallas_call(
        _tc_l1_body,
        grid=(_NB,),
        in_specs=[
            pl.BlockSpec((NC, _B, VP), lambda i: (0, i, 0)),
            pl.BlockSpec((_B, 1), lambda i: (i, 0)),
            pl.BlockSpec((_B, 1), lambda i: (i, 0)),
            pl.BlockSpec((VP, D), lambda i: (0, 0)),
            pl.BlockSpec((1, D), lambda i: (0, 0)),
        ],
        out_specs=pl.BlockSpec((_B, D), lambda i: (i, 0)),
        out_shape=jax.ShapeDtypeStruct((NP, D), f32),
    )(c_p, nodes_col, dinv_col, t1p, b1r)


def _tc_l2_body(sp_ref, y_ref, dinv_ref, bi_ref, w2_ref, b2_ref, out_ref,
                sums_scr, cnt_scr):
    i = pl.program_id(0)

    @pl.when(i == 0)
    def _init():
        sums_scr[...] = jnp.zeros((G, D), f32)
        cnt_scr[...] = jnp.zeros((G, D), f32)

    agg = dinv_ref[...] * (sp_ref[0] + sp_ref[1] + y_ref[...])
    x3 = jnp.maximum(
        jnp.dot(agg, w2_ref[...], preferred_element_type=f32) + b2_ref[...], 0.0
    )
    rows = i * _B + lax.broadcasted_iota(i32, (1, _B), 1)
    ohb = (lax.broadcasted_iota(i32, (G, _B), 0) == bi_ref[...]) & (rows < N)
    oh = jnp.where(ohb, 1.0, 0.0)
    sums_scr[...] += jnp.dot(oh, x3, preferred_element_type=f32)
    cnt_scr[...] += jnp.broadcast_to(jnp.sum(oh, axis=1, keepdims=True), (G, D))

    @pl.when(i == _NB - 1)
    def _fin():
        out_ref[...] = sums_scr[...] / jnp.maximum(cnt_scr[...], 1.0)


def _tc_layer2_pool(s_p, y, dinv_col, bi_row, w2, b2r):
    return pl.pallas_call(
        _tc_l2_body,
        grid=(_NB,),
        in_specs=[
            pl.BlockSpec((NC, _B, D), lambda i: (0, i, 0)),
            pl.BlockSpec((_B, D), lambda i: (i, 0)),
            pl.BlockSpec((_B, 1), lambda i: (i, 0)),
            pl.BlockSpec((1, _B), lambda i: (0, i)),
            pl.BlockSpec((D, D), lambda i: (0, 0)),
            pl.BlockSpec((1, D), lambda i: (0, 0)),
        ],
        out_specs=pl.BlockSpec((G, D), lambda i: (0, 0)),
        out_shape=jax.ShapeDtypeStruct((G, D), f32),
        scratch_shapes=[pltpu.VMEM((G, D), f32), pltpu.VMEM((G, D), f32)],
    )(s_p, y, dinv_col, bi_row, w2, b2r)


def kernel(nodes, edge_index, batch_index, emb, W1, b1, W2, b2):
    nodes = nodes.astype(i32)
    src = edge_index[0].astype(i32)
    dst = edge_index[1].astype(i32)
    bi = batch_index.astype(i32)

    nodes_pad = jnp.concatenate([nodes, jnp.zeros((NP - N,), i32)])
    bi_row = jnp.concatenate([bi, jnp.full((NP - N,), G - 1, i32)]).reshape(1, NP)
    emb_pad = jnp.zeros((VP, D), f32).at[: emb.shape[0]].set(emb)
    z1 = jnp.zeros((CZ // NS,), f32)
    z2 = jnp.zeros((NP, D), f32)

    deg_p = _sc_degree(dst, z1)
    dinv2d, t1p = _tc_prep(deg_p.reshape(NC, NP // 128, 128), emb_pad, W1)
    dinv_flat = dinv2d.reshape(NP)
    dinv_col = dinv2d.reshape(NP, 1)

    c_p = _sc_hist(src, dst, nodes_pad, dinv_flat, z1)
    y = _tc_layer1(
        c_p.reshape(NC, NP, VP), nodes_pad.reshape(NP, 1), dinv_col, t1p,
        b1.reshape(1, D),
    )
    s_p = _sc_segsum(src, dst, y, z2)
    return _tc_layer2_pool(s_p, y, dinv_col, bi_row, W2, b2.reshape(1, D))
